# baseline (speedup 1.0000x reference)
import jax
import jax.numpy as jnp
from jax.experimental import pallas as pl
from jax.experimental.pallas import tpu as pltpu

_EPS = 1e-5
_TOPKS = (192, 256, 288, 307)
_INT32_MIN = -2147483648


def _sortable_int(x):
    i = jax.lax.bitcast_convert_type(x, jnp.int32)
    return jnp.where(i < 0, jnp.int32(_INT32_MIN) - i, i)


def _kth_threshold(keys, k, lo, hi):
    def body(_, carry):
        lo, hi = carry
        mid = (lo & hi) + ((lo ^ hi) >> 1)
        cnt = jnp.sum((keys >= mid).astype(jnp.int32), axis=1, keepdims=True)
        pred = cnt >= k
        return jnp.where(pred, mid, lo), jnp.where(pred, hi, mid)

    lo, hi = jax.lax.fori_loop(0, 32, body, (lo, hi))
    return lo


def _body(x1_ref, x2_ref, ln1w_ref, ln1b_ref, rw_ref, rb_ref, ln2w_ref,
          ln2b_ref, coef_ref, out_ref):
    f32 = jnp.float32
    x1 = x1_ref[0]
    x2 = x2_ref[0]
    w1 = ln1w_ref[...]
    b1 = ln1b_ref[...]

    def ln_rows(x):
        mu = jnp.mean(x, axis=1, keepdims=True)
        xc = x - mu
        var = jnp.mean(xc * xc, axis=1, keepdims=True)
        return xc * jax.lax.rsqrt(var + _EPS) * w1 + b1

    n1 = ln_rows(x1)
    n2 = ln_rows(x2)

    ke = jnp.exp(n1 - jnp.max(n1, axis=0, keepdims=True))
    ks = ke / jnp.sum(ke, axis=0, keepdims=True)
    qe = jnp.exp(n2 - jnp.max(n2, axis=1, keepdims=True))
    qs = qe / jnp.sum(qe, axis=1, keepdims=True)

    ctxT = jax.lax.dot_general(n1, ks, (((0,), (0,)), ((), ())),
                               preferred_element_type=f32)

    D = ctxT.shape[0]
    keysT = _sortable_int(ctxT)
    m = jnp.max(ctxT, axis=0, keepdims=True)
    eT = jnp.exp(ctxT - m)

    lo0 = jnp.min(keysT, axis=0, keepdims=True)
    hi0 = jnp.max(keysT, axis=0, keepdims=True) + 1
    kvs = tuple(jnp.full((1, D), float(k), f32) for k in _TOPKS)
    ones8 = jnp.ones((8, D), f32)

    def bis_body(_, carry):
        los, his = carry
        nlos, nhis = [], []
        for i in range(4):
            lo, hi = los[i], his[i]
            mid = (lo & hi) + ((lo ^ hi) >> 1)
            maskf = jnp.where(keysT >= mid, 1.0, 0.0)
            cnt = jax.lax.dot_general(ones8, maskf, (((1,), (0,)), ((), ())),
                                      preferred_element_type=f32)[:1]
            pred = cnt >= kvs[i]
            nlos.append(jnp.where(pred, mid, lo))
            nhis.append(jnp.where(pred, hi, mid))
        return tuple(nlos), tuple(nhis)

    los, _ = jax.lax.fori_loop(0, 32, bis_body,
                               ((lo0,) * 4, (hi0,) * 4))

    coeffT = jnp.zeros_like(ctxT)
    for i in range(4):
        t = los[i]
        maskT = jnp.where(keysT >= t, 1.0, 0.0)
        s = jax.lax.dot_general(ones8, eT * maskT, (((1,), (0,)), ((), ())),
                                preferred_element_type=f32)[:1]
        coeffT += maskT * (coef_ref[i] / s)

    wmatT = eT * coeffT
    att = jax.lax.dot_general(wmatT, qs, (((0,), (1,)), ((), ())),
                              preferred_element_type=f32)
    outp = jax.lax.dot_general(rw_ref[...], att, (((1,), (0,)), ((), ())),
                               preferred_element_type=f32)
    outp = outp + rb_ref[...]

    mu = jnp.mean(outp, axis=0, keepdims=True)
    oc = outp - mu
    var = jnp.mean(oc * oc, axis=0, keepdims=True)
    out_ref[0] = oc * jax.lax.rsqrt(var + _EPS) * ln2w_ref[...] + ln2b_ref[...]


def _build(B, N, C, interpret=False):
    return pl.pallas_call(
        _body,
        grid=(B,),
        in_specs=[
            pl.BlockSpec((1, N, C), lambda b: (b, 0, 0)),
            pl.BlockSpec((1, N, C), lambda b: (b, 0, 0)),
            pl.BlockSpec((1, C), lambda b: (0, 0)),
            pl.BlockSpec((1, C), lambda b: (0, 0)),
            pl.BlockSpec((2 * C, C), lambda b: (0, 0)),
            pl.BlockSpec((2 * C, 1), lambda b: (0, 0)),
            pl.BlockSpec((2 * C, 1), lambda b: (0, 0)),
            pl.BlockSpec((2 * C, 1), lambda b: (0, 0)),
            pl.BlockSpec(memory_space=pltpu.SMEM),
        ],
        out_specs=pl.BlockSpec((1, 2 * C, N), lambda b: (b, 0, 0)),
        out_shape=jax.ShapeDtypeStruct((B, 2 * C, N), jnp.float32),
        interpret=interpret,
    )


def kernel(x1, x2, ln1_w, ln1_b, reproj_w, reproj_b, ln2_w, ln2_b,
           a1, a2, a3, a4):
    B, H, W, C = x1.shape
    N = H * W
    x1f = x1.reshape(B, N, C)
    x2f = x2.reshape(B, N, C)
    coefs = jnp.concatenate([a1, a2, a3, a4]).astype(jnp.float32)
    out = _build(B, N, C)(
        x1f, x2f, ln1_w.reshape(1, C), ln1_b.reshape(1, C), reproj_w,
        reproj_b.reshape(2 * C, 1), ln2_w.reshape(2 * C, 1),
        ln2_b.reshape(2 * C, 1), coefs)
    return out.reshape(B, 2 * C, H, W)


# speedup vs baseline: 1.2420x; 1.2420x over previous
import jax
import jax.numpy as jnp
from jax.experimental import pallas as pl
from jax.experimental.pallas import tpu as pltpu

_EPS = 1e-5
_TOPKS = (192, 256, 288, 307)
_INT32_MIN = -2147483648


def _sortable_int(x):
    i = jax.lax.bitcast_convert_type(x, jnp.int32)
    return jnp.where(i < 0, jnp.int32(_INT32_MIN) - i, i)


def _kth_threshold(keys, k, lo, hi):
    def body(_, carry):
        lo, hi = carry
        mid = (lo & hi) + ((lo ^ hi) >> 1)
        cnt = jnp.sum((keys >= mid).astype(jnp.int32), axis=1, keepdims=True)
        pred = cnt >= k
        return jnp.where(pred, mid, lo), jnp.where(pred, hi, mid)

    lo, hi = jax.lax.fori_loop(0, 32, body, (lo, hi))
    return lo


def _body(x1_ref, x2_ref, ln1w_ref, ln1b_ref, rw_ref, rb_ref, ln2w_ref,
          ln2b_ref, coef_ref, out_ref):
    f32 = jnp.float32
    x1 = x1_ref[0]
    x2 = x2_ref[0]
    w1 = ln1w_ref[...]
    b1 = ln1b_ref[...]

    def ln_rows(x):
        mu = jnp.mean(x, axis=1, keepdims=True)
        xc = x - mu
        var = jnp.mean(xc * xc, axis=1, keepdims=True)
        return xc * jax.lax.rsqrt(var + _EPS) * w1 + b1

    n1 = ln_rows(x1)
    n2 = ln_rows(x2)

    ke = jnp.exp(n1 - jnp.max(n1, axis=0, keepdims=True))
    ks = ke / jnp.sum(ke, axis=0, keepdims=True)
    qe = jnp.exp(n2 - jnp.max(n2, axis=1, keepdims=True))
    qs = qe / jnp.sum(qe, axis=1, keepdims=True)

    ctxT = jax.lax.dot_general(n1, ks, (((0,), (0,)), ((), ())),
                               preferred_element_type=f32)

    D = ctxT.shape[0]
    keysT = _sortable_int(ctxT)
    m = jnp.max(ctxT, axis=0, keepdims=True)
    eT = jnp.exp(ctxT - m)

    keysT4 = jnp.concatenate([keysT, keysT, keysT, keysT], axis=1)
    lo = jnp.min(keysT4, axis=0, keepdims=True)
    hi = jnp.max(keysT4, axis=0, keepdims=True) + 1
    lane = jax.lax.broadcasted_iota(jnp.int32, (1, 4 * D), 1) // D
    kv = jnp.where(lane == 0, _TOPKS[0],
                   jnp.where(lane == 1, _TOPKS[1],
                             jnp.where(lane == 2, _TOPKS[2], _TOPKS[3])))
    kv = kv.astype(f32)
    ones8 = jnp.ones((8, D), f32)

    for _ in range(32):
        mid = (lo & hi) + ((lo ^ hi) >> 1)
        maskf = jnp.where(keysT4 >= mid, 1.0, 0.0)
        cnt = jax.lax.dot_general(ones8, maskf, (((1,), (0,)), ((), ())),
                                  preferred_element_type=f32)[:1]
        pred = cnt >= kv
        lo = jnp.where(pred, mid, lo)
        hi = jnp.where(pred, hi, mid)

    coeffT = jnp.zeros_like(ctxT)
    for i in range(4):
        t = lo[:, i * D:(i + 1) * D]
        maskT = jnp.where(keysT >= t, 1.0, 0.0)
        s = jax.lax.dot_general(ones8, eT * maskT, (((1,), (0,)), ((), ())),
                                preferred_element_type=f32)[:1]
        coeffT += maskT * (coef_ref[i] / s)

    wmatT = eT * coeffT
    att = jax.lax.dot_general(wmatT, qs, (((0,), (1,)), ((), ())),
                              preferred_element_type=f32)
    outp = jax.lax.dot_general(rw_ref[...], att, (((1,), (0,)), ((), ())),
                               preferred_element_type=f32)
    outp = outp + rb_ref[...]

    mu = jnp.mean(outp, axis=0, keepdims=True)
    oc = outp - mu
    var = jnp.mean(oc * oc, axis=0, keepdims=True)
    out_ref[0] = oc * jax.lax.rsqrt(var + _EPS) * ln2w_ref[...] + ln2b_ref[...]


def _build(B, N, C, interpret=False):
    return pl.pallas_call(
        _body,
        grid=(B,),
        in_specs=[
            pl.BlockSpec((1, N, C), lambda b: (b, 0, 0)),
            pl.BlockSpec((1, N, C), lambda b: (b, 0, 0)),
            pl.BlockSpec((1, C), lambda b: (0, 0)),
            pl.BlockSpec((1, C), lambda b: (0, 0)),
            pl.BlockSpec((2 * C, C), lambda b: (0, 0)),
            pl.BlockSpec((2 * C, 1), lambda b: (0, 0)),
            pl.BlockSpec((2 * C, 1), lambda b: (0, 0)),
            pl.BlockSpec((2 * C, 1), lambda b: (0, 0)),
            pl.BlockSpec(memory_space=pltpu.SMEM),
        ],
        out_specs=pl.BlockSpec((1, 2 * C, N), lambda b: (b, 0, 0)),
        out_shape=jax.ShapeDtypeStruct((B, 2 * C, N), jnp.float32),
        interpret=interpret,
    )


def kernel(x1, x2, ln1_w, ln1_b, reproj_w, reproj_b, ln2_w, ln2_b,
           a1, a2, a3, a4):
    B, H, W, C = x1.shape
    N = H * W
    x1f = x1.reshape(B, N, C)
    x2f = x2.reshape(B, N, C)
    coefs = jnp.concatenate([a1, a2, a3, a4]).astype(jnp.float32)
    out = _build(B, N, C)(
        x1f, x2f, ln1_w.reshape(1, C), ln1_b.reshape(1, C), reproj_w,
        reproj_b.reshape(2 * C, 1), ln2_w.reshape(2 * C, 1),
        ln2_b.reshape(2 * C, 1), coefs)
    return out.reshape(B, 2 * C, H, W)


# speedup vs baseline: 1.2464x; 1.0036x over previous
import jax
import jax.numpy as jnp
from jax.experimental import pallas as pl
from jax.experimental.pallas import tpu as pltpu

_EPS = 1e-5
_TOPKS = (192, 256, 288, 307)
_INT32_MIN = -2147483648


def _sortable_int(x):
    i = jax.lax.bitcast_convert_type(x, jnp.int32)
    return jnp.where(i < 0, jnp.int32(_INT32_MIN) - i, i)


def _kth_threshold(keys, k, lo, hi):
    def body(_, carry):
        lo, hi = carry
        mid = (lo & hi) + ((lo ^ hi) >> 1)
        cnt = jnp.sum((keys >= mid).astype(jnp.int32), axis=1, keepdims=True)
        pred = cnt >= k
        return jnp.where(pred, mid, lo), jnp.where(pred, hi, mid)

    lo, hi = jax.lax.fori_loop(0, 32, body, (lo, hi))
    return lo


def _body(x1_ref, x2_ref, ln1w_ref, ln1b_ref, rw_ref, rb_ref, ln2w_ref,
          ln2b_ref, a1_ref, a2_ref, a3_ref, a4_ref, out_ref):
    coef_refs = (a1_ref, a2_ref, a3_ref, a4_ref)
    f32 = jnp.float32
    x1 = x1_ref[0]
    x2 = x2_ref[0]
    w1 = ln1w_ref[...]
    b1 = ln1b_ref[...]

    def ln_rows(x):
        mu = jnp.mean(x, axis=1, keepdims=True)
        xc = x - mu
        var = jnp.mean(xc * xc, axis=1, keepdims=True)
        return xc * jax.lax.rsqrt(var + _EPS) * w1 + b1

    n1 = ln_rows(x1)
    n2 = ln_rows(x2)

    ke = jnp.exp(n1 - jnp.max(n1, axis=0, keepdims=True))
    ks = ke / jnp.sum(ke, axis=0, keepdims=True)
    qe = jnp.exp(n2 - jnp.max(n2, axis=1, keepdims=True))
    qs = qe / jnp.sum(qe, axis=1, keepdims=True)

    ctxT = jax.lax.dot_general(n1, ks, (((0,), (0,)), ((), ())),
                               preferred_element_type=f32)

    D = ctxT.shape[0]
    keysT = _sortable_int(ctxT)
    m = jnp.max(ctxT, axis=0, keepdims=True)
    eT = jnp.exp(ctxT - m)

    keysT4 = jnp.concatenate([keysT, keysT, keysT, keysT], axis=1)
    lo = jnp.min(keysT4, axis=0, keepdims=True)
    hi = jnp.max(keysT4, axis=0, keepdims=True) + 1
    lane = jax.lax.broadcasted_iota(jnp.int32, (1, 4 * D), 1) // D
    kv = jnp.where(lane == 0, _TOPKS[0],
                   jnp.where(lane == 1, _TOPKS[1],
                             jnp.where(lane == 2, _TOPKS[2], _TOPKS[3])))
    kv = kv.astype(f32)
    ones8 = jnp.ones((8, D), f32)

    for _ in range(32):
        mid = (lo & hi) + ((lo ^ hi) >> 1)
        maskf = jnp.where(keysT4 >= mid, 1.0, 0.0)
        cnt = jax.lax.dot_general(ones8, maskf, (((1,), (0,)), ((), ())),
                                  preferred_element_type=f32)[:1]
        pred = cnt >= kv
        lo = jnp.where(pred, mid, lo)
        hi = jnp.where(pred, hi, mid)

    coeffT = jnp.zeros_like(ctxT)
    for i in range(4):
        t = lo[:, i * D:(i + 1) * D]
        maskT = jnp.where(keysT >= t, 1.0, 0.0)
        s = jax.lax.dot_general(ones8, eT * maskT, (((1,), (0,)), ((), ())),
                                preferred_element_type=f32)[:1]
        coeffT += maskT * (coef_refs[i][0] / s)

    wmatT = eT * coeffT
    att = jax.lax.dot_general(wmatT, qs, (((0,), (1,)), ((), ())),
                              preferred_element_type=f32)
    outp = jax.lax.dot_general(rw_ref[...], att, (((1,), (0,)), ((), ())),
                               preferred_element_type=f32)
    outp = outp + rb_ref[...]

    mu = jnp.mean(outp, axis=0, keepdims=True)
    oc = outp - mu
    var = jnp.mean(oc * oc, axis=0, keepdims=True)
    out_ref[0] = oc * jax.lax.rsqrt(var + _EPS) * ln2w_ref[...] + ln2b_ref[...]


def _build(B, N, C, interpret=False):
    return pl.pallas_call(
        _body,
        grid=(B,),
        in_specs=[
            pl.BlockSpec((1, N, C), lambda b: (b, 0, 0)),
            pl.BlockSpec((1, N, C), lambda b: (b, 0, 0)),
            pl.BlockSpec((1, C), lambda b: (0, 0)),
            pl.BlockSpec((1, C), lambda b: (0, 0)),
            pl.BlockSpec((2 * C, C), lambda b: (0, 0)),
            pl.BlockSpec((2 * C, 1), lambda b: (0, 0)),
            pl.BlockSpec((2 * C, 1), lambda b: (0, 0)),
            pl.BlockSpec((2 * C, 1), lambda b: (0, 0)),
            pl.BlockSpec(memory_space=pltpu.SMEM),
            pl.BlockSpec(memory_space=pltpu.SMEM),
            pl.BlockSpec(memory_space=pltpu.SMEM),
            pl.BlockSpec(memory_space=pltpu.SMEM),
        ],
        out_specs=pl.BlockSpec((1, 2 * C, N), lambda b: (b, 0, 0)),
        out_shape=jax.ShapeDtypeStruct((B, 2 * C, N), jnp.float32),
        interpret=interpret,
    )


def kernel(x1, x2, ln1_w, ln1_b, reproj_w, reproj_b, ln2_w, ln2_b,
           a1, a2, a3, a4):
    B, H, W, C = x1.shape
    N = H * W
    x1f = x1.reshape(B, N, C)
    x2f = x2.reshape(B, N, C)
    out = _build(B, N, C)(
        x1f, x2f, ln1_w.reshape(1, C), ln1_b.reshape(1, C), reproj_w,
        reproj_b.reshape(2 * C, 1), ln2_w.reshape(2 * C, 1),
        ln2_b.reshape(2 * C, 1), a1, a2, a3, a4)
    return out.reshape(B, 2 * C, H, W)


# speedup vs baseline: 1.3251x; 1.0632x over previous
import jax
import jax.numpy as jnp
from jax.experimental import pallas as pl
from jax.experimental.pallas import tpu as pltpu

_EPS = 1e-5
_TOPKS = (192, 256, 288, 307)
_INT32_MIN = -2147483648


def _sortable_int(x):
    i = jax.lax.bitcast_convert_type(x, jnp.int32)
    return jnp.where(i < 0, jnp.int32(_INT32_MIN) - i, i)


def _kth_threshold(keys, k, lo, hi):
    def body(_, carry):
        lo, hi = carry
        mid = (lo & hi) + ((lo ^ hi) >> 1)
        cnt = jnp.sum((keys >= mid).astype(jnp.int32), axis=1, keepdims=True)
        pred = cnt >= k
        return jnp.where(pred, mid, lo), jnp.where(pred, hi, mid)

    lo, hi = jax.lax.fori_loop(0, 32, body, (lo, hi))
    return lo


def _body(x1_ref, x2_ref, ln1w_ref, ln1b_ref, rw_ref, rb_ref, ln2w_ref,
          ln2b_ref, a1_ref, a2_ref, a3_ref, a4_ref, out_ref):
    coef_refs = (a1_ref, a2_ref, a3_ref, a4_ref)
    f32 = jnp.float32
    x1 = x1_ref[0]
    x2 = x2_ref[0]
    w1 = ln1w_ref[...].reshape(1, -1)
    b1 = ln1b_ref[...].reshape(1, -1)

    def ln_rows(x):
        mu = jnp.mean(x, axis=1, keepdims=True)
        xc = x - mu
        var = jnp.mean(xc * xc, axis=1, keepdims=True)
        return xc * jax.lax.rsqrt(var + _EPS) * w1 + b1

    n1 = ln_rows(x1)
    n2 = ln_rows(x2)

    ke = jnp.exp(n1 - jnp.max(n1, axis=0, keepdims=True))
    ks = ke / jnp.sum(ke, axis=0, keepdims=True)
    qe = jnp.exp(n2 - jnp.max(n2, axis=1, keepdims=True))
    qs = qe / jnp.sum(qe, axis=1, keepdims=True)

    ctxT = jax.lax.dot_general(n1, ks, (((0,), (0,)), ((), ())),
                               preferred_element_type=f32)

    D = ctxT.shape[0]
    keysT = _sortable_int(ctxT)
    m = jnp.max(ctxT, axis=0, keepdims=True)
    eT = jnp.exp(ctxT - m)

    keysT4 = jnp.concatenate([keysT, keysT, keysT, keysT], axis=1)
    lo = jnp.min(keysT4, axis=0, keepdims=True)
    hi = jnp.max(keysT4, axis=0, keepdims=True) + 1
    lane = jax.lax.broadcasted_iota(jnp.int32, (1, 4 * D), 1) // D
    kv = jnp.where(lane == 0, _TOPKS[0],
                   jnp.where(lane == 1, _TOPKS[1],
                             jnp.where(lane == 2, _TOPKS[2], _TOPKS[3])))
    kv = kv.astype(f32)
    ones8 = jnp.ones((8, D), f32)

    for _ in range(32):
        mid = (lo & hi) + ((lo ^ hi) >> 1)
        maskf = jnp.where(keysT4 >= mid, 1.0, 0.0)
        cnt = jax.lax.dot_general(ones8, maskf, (((1,), (0,)), ((), ())),
                                  preferred_element_type=f32)[:1]
        pred = cnt >= kv
        lo = jnp.where(pred, mid, lo)
        hi = jnp.where(pred, hi, mid)

    coeffT = jnp.zeros_like(ctxT)
    for i in range(4):
        t = lo[:, i * D:(i + 1) * D]
        maskT = jnp.where(keysT >= t, 1.0, 0.0)
        s = jax.lax.dot_general(ones8, eT * maskT, (((1,), (0,)), ((), ())),
                                preferred_element_type=f32)[:1]
        coeffT += maskT * (coef_refs[i][0] / s)

    wmatT = eT * coeffT
    att = jax.lax.dot_general(wmatT, qs, (((0,), (1,)), ((), ())),
                              preferred_element_type=f32)
    rows8 = jnp.concatenate(
        [rb_ref[...].reshape(1, -1), ln2w_ref[...].reshape(1, -1),
         ln2b_ref[...].reshape(1, -1)] + [jnp.zeros((5, 2 * D), f32)], axis=0)
    cols = jax.lax.transpose(rows8, (1, 0))
    rb_c = cols[:, 0:1]
    w2_c = cols[:, 1:2]
    b2_c = cols[:, 2:3]

    outp = jax.lax.dot_general(rw_ref[...], att, (((1,), (0,)), ((), ())),
                               preferred_element_type=f32)
    outp = outp + rb_c

    mu = jnp.mean(outp, axis=0, keepdims=True)
    oc = outp - mu
    var = jnp.mean(oc * oc, axis=0, keepdims=True)
    out_ref[0] = oc * jax.lax.rsqrt(var + _EPS) * w2_c + b2_c


def _build(B, N, C, interpret=False):
    return pl.pallas_call(
        _body,
        grid=(B,),
        in_specs=[
            pl.BlockSpec((1, N, C), lambda b: (b, 0, 0)),
            pl.BlockSpec((1, N, C), lambda b: (b, 0, 0)),
            pl.BlockSpec((C,), lambda b: (0,)),
            pl.BlockSpec((C,), lambda b: (0,)),
            pl.BlockSpec((2 * C, C), lambda b: (0, 0)),
            pl.BlockSpec((2 * C,), lambda b: (0,)),
            pl.BlockSpec((2 * C,), lambda b: (0,)),
            pl.BlockSpec((2 * C,), lambda b: (0,)),
            pl.BlockSpec(memory_space=pltpu.SMEM),
            pl.BlockSpec(memory_space=pltpu.SMEM),
            pl.BlockSpec(memory_space=pltpu.SMEM),
            pl.BlockSpec(memory_space=pltpu.SMEM),
        ],
        out_specs=pl.BlockSpec((1, 2 * C, N), lambda b: (b, 0, 0)),
        out_shape=jax.ShapeDtypeStruct((B, 2 * C, N), jnp.float32),
        interpret=interpret,
    )


def kernel(x1, x2, ln1_w, ln1_b, reproj_w, reproj_b, ln2_w, ln2_b,
           a1, a2, a3, a4):
    B, H, W, C = x1.shape
    N = H * W
    x1f = x1.reshape(B, N, C)
    x2f = x2.reshape(B, N, C)
    out = _build(B, N, C)(
        x1f, x2f, ln1_w, ln1_b, reproj_w,
        reproj_b, ln2_w, ln2_b, a1, a2, a3, a4)
    return out.reshape(B, 2 * C, H, W)
